# baseline (speedup 1.0000x reference)
import functools

import jax
import jax.numpy as jnp
from jax import lax
from jax.experimental import pallas as pl
from jax.experimental.pallas import tpu as pltpu
from jax.experimental.pallas import tpu_sc as plsc

BATCH = 16384
D = 64

_NC, _NS = 2, 16
_NW = _NC * _NS
_B_PER_W = BATCH // _NW
_CHUNK = 128
_NCHUNK = _B_PER_W // _CHUNK


@functools.cache
def _make_sc_gather():
    mesh = plsc.VectorSubcoreMesh(core_axis_name="c", subcore_axis_name="s")

    @functools.partial(
        pl.kernel,
        out_type=(
            jax.ShapeDtypeStruct((BATCH, D), jnp.float32),
            jax.ShapeDtypeStruct((BATCH, D), jnp.float32),
        ),
        mesh=mesh,
        scratch_types=[
            pltpu.VMEM((_NCHUNK, _CHUNK), jnp.int32),
            pltpu.VMEM((_NCHUNK, _CHUNK), jnp.int32),
            pltpu.VMEM((_B_PER_W, D), jnp.float32),
            pltpu.VMEM((_B_PER_W, D), jnp.float32),
            pltpu.SemaphoreType.DMA,
        ],
        compiler_params=pltpu.CompilerParams(use_tc_tiling_on_sc=False),
    )
    def sc_gather(ut_hbm, bt_hbm, uidx_hbm, bidx_hbm, ue_hbm, be_hbm,
                  uidx_v, bidx_v, urows_v, brows_v, sem):
        wid = lax.axis_index("s") * _NC + lax.axis_index("c")
        base = wid * _B_PER_W
        pltpu.sync_copy(uidx_hbm.at[wid], uidx_v)
        pltpu.sync_copy(bidx_hbm.at[wid], bidx_v)
        descs = []
        for j in range(_NCHUNK):
            descs.append(pltpu.async_copy(
                ut_hbm.at[uidx_v.at[j]],
                urows_v.at[pl.ds(j * _CHUNK, _CHUNK)], sem))
            descs.append(pltpu.async_copy(
                bt_hbm.at[bidx_v.at[j]],
                brows_v.at[pl.ds(j * _CHUNK, _CHUNK)], sem))
        for d in descs:
            d.wait()
        pltpu.sync_copy(urows_v, ue_hbm.at[pl.ds(base, _B_PER_W)])
        pltpu.sync_copy(brows_v, be_hbm.at[pl.ds(base, _B_PER_W)])

    return sc_gather


_BM = 512


def _mlp_body(ue_ref, be_ref, w1a_ref, w1b_ref, b1_ref, w2_ref, b2_ref,
              w3_ref, b3_ref, w4_ref, b4_ref, out_ref):
    h = ue_ref[...] @ w1a_ref[...] + be_ref[...] @ w1b_ref[...] + b1_ref[...]
    h = jnp.maximum(h, 0.0)
    h = jnp.maximum(h @ w2_ref[...] + b2_ref[...], 0.0)
    h = jnp.maximum(h @ w3_ref[...] + b3_ref[...], 0.0)
    out_ref[...] = jnp.sum(h * w4_ref[...], axis=1) + b4_ref[0]


def _mlp(ue, be, W1a, W1b, b1, W2, b2, W3, b3, w4row, b4):
    grid = (BATCH // _BM,)
    full = lambda i: (0, 0)
    return pl.pallas_call(
        _mlp_body,
        grid=grid,
        in_specs=[
            pl.BlockSpec((_BM, D), lambda i: (i, 0)),
            pl.BlockSpec((_BM, D), lambda i: (i, 0)),
            pl.BlockSpec((D, 1024), full),
            pl.BlockSpec((D, 1024), full),
            pl.BlockSpec((1, 1024), full),
            pl.BlockSpec((1024, 512), full),
            pl.BlockSpec((1, 512), full),
            pl.BlockSpec((512, 256), full),
            pl.BlockSpec((1, 256), full),
            pl.BlockSpec((1, 256), full),
            pl.BlockSpec(memory_space=pltpu.SMEM),
        ],
        out_specs=pl.BlockSpec((_BM,), lambda i: (i,)),
        out_shape=jax.ShapeDtypeStruct((BATCH,), jnp.float32),
    )(ue, be, W1a, W1b, b1, W2, b2, W3, b3, w4row, b4)


def kernel(users, businesses, user_table, business_table,
           W1, b1, W2, b2, W3, b3, W4, b4):
    uidx = users.astype(jnp.int32).reshape(_NW, _NCHUNK, _CHUNK)
    bidx = businesses.astype(jnp.int32).reshape(_NW, _NCHUNK, _CHUNK)
    ue, be = _make_sc_gather()(user_table, business_table, uidx, bidx)
    W1a = W1[:D]
    W1b = W1[D:]
    w4row = W4.reshape(1, 256)
    return _mlp(ue, be, W1a, W1b, b1.reshape(1, 1024), W2, b2.reshape(1, 512),
                W3, b3.reshape(1, 256), w4row, b4)


# speedup vs baseline: 1.0038x; 1.0038x over previous
import functools

import jax
import jax.numpy as jnp
from jax import lax
from jax.experimental import pallas as pl
from jax.experimental.pallas import tpu as pltpu
from jax.experimental.pallas import tpu_sc as plsc

BATCH = 16384
D = 64

_NC, _NS = 2, 16
_NW = _NC * _NS
_B_PER_W = BATCH // _NW
_CHUNK = 128
_NCHUNK = _B_PER_W // _CHUNK


@functools.cache
def _make_sc_gather():
    mesh = plsc.VectorSubcoreMesh(core_axis_name="c", subcore_axis_name="s")

    @functools.partial(
        pl.kernel,
        out_type=(
            jax.ShapeDtypeStruct((BATCH, D), jnp.float32),
            jax.ShapeDtypeStruct((BATCH, D), jnp.float32),
        ),
        mesh=mesh,
        scratch_types=[
            pltpu.VMEM((_NCHUNK, _CHUNK), jnp.int32),
            pltpu.VMEM((_NCHUNK, _CHUNK), jnp.int32),
            pltpu.VMEM((_B_PER_W, D), jnp.float32),
            pltpu.VMEM((_B_PER_W, D), jnp.float32),
            pltpu.SemaphoreType.DMA,
        ],
        compiler_params=pltpu.CompilerParams(use_tc_tiling_on_sc=False),
    )
    def sc_gather(ut_hbm, bt_hbm, uidx_hbm, bidx_hbm, ue_hbm, be_hbm,
                  uidx_v, bidx_v, urows_v, brows_v, sem):
        wid = lax.axis_index("s") * _NC + lax.axis_index("c")
        base = wid * _B_PER_W
        pltpu.sync_copy(uidx_hbm.at[wid], uidx_v)
        pltpu.sync_copy(bidx_hbm.at[wid], bidx_v)
        descs = []
        for j in range(_NCHUNK):
            descs.append(pltpu.async_copy(
                ut_hbm.at[uidx_v.at[j]],
                urows_v.at[pl.ds(j * _CHUNK, _CHUNK)], sem))
            descs.append(pltpu.async_copy(
                bt_hbm.at[bidx_v.at[j]],
                brows_v.at[pl.ds(j * _CHUNK, _CHUNK)], sem))
        for d in descs:
            d.wait()
        pltpu.sync_copy(urows_v, ue_hbm.at[pl.ds(base, _B_PER_W)])
        pltpu.sync_copy(brows_v, be_hbm.at[pl.ds(base, _B_PER_W)])

    return sc_gather


_BM = 512


def _dot(a, b):
    return jax.lax.dot(a, b, preferred_element_type=jnp.float32)


def _mlp_body(ue_ref, be_ref, w1a_ref, w1b_ref, b1_ref, w2_ref, b2_ref,
              w3_ref, b3_ref, w4_ref, b4_ref, out_ref):
    ue = ue_ref[...].astype(jnp.bfloat16)
    be = be_ref[...].astype(jnp.bfloat16)
    h = _dot(ue, w1a_ref[...]) + _dot(be, w1b_ref[...]) + b1_ref[...]
    h = jnp.maximum(h, 0.0).astype(jnp.bfloat16)
    h = jnp.maximum(_dot(h, w2_ref[...]) + b2_ref[...], 0.0).astype(jnp.bfloat16)
    h = jnp.maximum(_dot(h, w3_ref[...]) + b3_ref[...], 0.0)
    out_ref[...] = jnp.sum(h * w4_ref[...], axis=1) + b4_ref[0]


def _mlp(ue, be, W1a, W1b, b1, W2, b2, W3, b3, w4row, b4):
    grid = (BATCH // _BM,)
    full = lambda i: (0, 0)
    return pl.pallas_call(
        _mlp_body,
        grid=grid,
        in_specs=[
            pl.BlockSpec((_BM, D), lambda i: (i, 0)),
            pl.BlockSpec((_BM, D), lambda i: (i, 0)),
            pl.BlockSpec((D, 1024), full),
            pl.BlockSpec((D, 1024), full),
            pl.BlockSpec((1, 1024), full),
            pl.BlockSpec((1024, 512), full),
            pl.BlockSpec((1, 512), full),
            pl.BlockSpec((512, 256), full),
            pl.BlockSpec((1, 256), full),
            pl.BlockSpec((1, 256), full),
            pl.BlockSpec(memory_space=pltpu.SMEM),
        ],
        out_specs=pl.BlockSpec((_BM,), lambda i: (i,)),
        out_shape=jax.ShapeDtypeStruct((BATCH,), jnp.float32),
    )(ue, be, W1a, W1b, b1, W2, b2, W3, b3, w4row, b4)


def kernel(users, businesses, user_table, business_table,
           W1, b1, W2, b2, W3, b3, W4, b4):
    uidx = users.astype(jnp.int32).reshape(_NW, _NCHUNK, _CHUNK)
    bidx = businesses.astype(jnp.int32).reshape(_NW, _NCHUNK, _CHUNK)
    ue, be = _make_sc_gather()(user_table, business_table, uidx, bidx)
    W1a = W1[:D].astype(jnp.bfloat16)
    W1b = W1[D:].astype(jnp.bfloat16)
    w4row = W4.reshape(1, 256)
    return _mlp(ue, be, W1a, W1b, b1.reshape(1, 1024),
                W2.astype(jnp.bfloat16), b2.reshape(1, 512),
                W3.astype(jnp.bfloat16), b3.reshape(1, 256), w4row, b4)
